# baseline (speedup 1.0000x reference)
import functools

import jax
import jax.numpy as jnp
from jax import lax
from jax.experimental import pallas as pl
from jax.experimental.pallas import tpu as pltpu
from jax.experimental.pallas import tpu_sc as plsc

B, T, C, H, E = 1, 2048, 768, 3072, 8
TB = 128
NB = T // TB + E
NPAD = NB * TB
NW = 32
BPW = NPAD // NW
NCHUNK = 6
CH = BPW // NCHUNK


def _gating_body(x_ref, wred_ref, wg_ref, score_ref, idx_ref):
    xf = x_ref[...]
    red = jnp.dot(xf, wred_ref[...].T,
                  preferred_element_type=jnp.float32)
    wg = wg_ref[...]
    norm = jnp.sqrt(jnp.sum(wg * wg, axis=1, keepdims=True))
    wg_s = wg * (1.5 / norm)
    n2 = jnp.sqrt(jnp.sum(wg_s * wg_s, axis=1, keepdims=True))
    wg_n = wg_s / jnp.maximum(n2, 1e-4)
    logits = jnp.dot(red, wg_n.T,
                     preferred_element_type=jnp.float32)
    lmax = jnp.max(logits, axis=1, keepdims=True)
    z = jnp.sum(jnp.exp(logits - lmax), axis=1, keepdims=True)
    score_ref[...] = 1.0 / z
    col = lax.broadcasted_iota(jnp.int32, logits.shape, 1)
    idx_ref[...] = jnp.min(
        jnp.where(logits >= lmax, col, jnp.int32(E)), axis=1, keepdims=True
    )


def _gating_tc(xf, Wred, wg):
    return pl.pallas_call(
        _gating_body,
        out_shape=(
            jax.ShapeDtypeStruct((T, 1), jnp.float32),
            jax.ShapeDtypeStruct((T, 1), jnp.int32),
        ),
    )(xf, Wred, wg)


def _sc_gather_body(x_hbm, src_hbm, out_hbm, idx_v, rows_v, sem):
    wid = lax.axis_index("s") * 2 + lax.axis_index("c")
    base = wid * BPW
    pltpu.sync_copy(src_hbm.at[pl.ds(base, BPW)], idx_v)
    cps = [
        pltpu.async_copy(x_hbm.at[idx_v.at[pl.ds(k * CH, CH)]],
                         rows_v.at[pl.ds(k * CH, CH)], sem)
        for k in range(NCHUNK)
    ]
    for c in cps:
        c.wait()
    pltpu.sync_copy(rows_v, out_hbm.at[pl.ds(base, BPW)])


def _sc_gather(xf, src_ids):
    mesh = plsc.VectorSubcoreMesh(core_axis_name="c", subcore_axis_name="s")
    return pl.kernel(
        _sc_gather_body,
        out_type=jax.ShapeDtypeStruct((NPAD, C), jnp.float32),
        mesh=mesh,
        scratch_types=[
            pltpu.VMEM((BPW,), jnp.int32),
            pltpu.VMEM((BPW, C), jnp.float32),
            pltpu.SemaphoreType.DMA,
        ],
    )(xf, src_ids)


def _sc_scatter_body(rows_hbm, dst_hbm, out_hbm, idx_v, rows_v, sem, semw):
    wid = lax.axis_index("s") * 2 + lax.axis_index("c")
    base = wid * BPW
    pltpu.sync_copy(dst_hbm.at[wid], idx_v)
    pltpu.async_copy(rows_hbm.at[pl.ds(base, BPW)], rows_v, sem).wait()
    cps = [
        pltpu.async_copy(rows_v.at[pl.ds(k * CH, CH)],
                         out_hbm.at[idx_v.at[k]], semw)
        for k in range(NCHUNK)
    ]
    for c in cps:
        c.wait()


def _sc_scatter(rows, dst_ids):
    mesh = plsc.VectorSubcoreMesh(core_axis_name="c", subcore_axis_name="s")
    return pl.kernel(
        _sc_scatter_body,
        out_type=jax.ShapeDtypeStruct((T + 8, C), jnp.float32),
        mesh=mesh,
        scratch_types=[
            pltpu.VMEM((NCHUNK, CH), jnp.int32),
            pltpu.VMEM((BPW, C), jnp.float32),
            pltpu.SemaphoreType.DMA,
            pltpu.SemaphoreType.DMA,
        ],
    )(rows, dst_ids.reshape(NW, NCHUNK, CH))


def _ffn_body(ex_ref, xs_ref, sc_ref, w1_ref, b1_ref, w2_ref, b2_ref,
              ys_ref, sum_ref):
    u = pl.program_id(0)
    xt = xs_ref[...].astype(jnp.bfloat16)
    h = jnp.dot(xt, w1_ref[0].astype(jnp.bfloat16),
                preferred_element_type=jnp.float32)
    h = h + b1_ref[0]
    h = 0.5 * h * (1.0 + lax.erf(h * 0.7071067811865476))
    y = jnp.dot(h.astype(jnp.bfloat16), w2_ref[0].astype(jnp.bfloat16),
                preferred_element_type=jnp.float32)
    y = (y + b2_ref[0]) * sc_ref[...]
    ys_ref[...] = y

    @pl.when(u == 0)
    def _():
        sum_ref[...] = jnp.zeros((1, 1), jnp.float32)

    sum_ref[...] += jnp.sum(y).reshape(1, 1)


def _ffn_tc(xs, scores_pad, unit_expert, W1, b1, W2, b2):
    grid_spec = pltpu.PrefetchScalarGridSpec(
        num_scalar_prefetch=1,
        grid=(NB,),
        in_specs=[
            pl.BlockSpec((TB, C), lambda u, ex: (u, 0)),
            pl.BlockSpec((TB, 1), lambda u, ex: (u, 0)),
            pl.BlockSpec((1, C, H), lambda u, ex: (ex[u], 0, 0)),
            pl.BlockSpec((1, 1, H), lambda u, ex: (ex[u], 0, 0)),
            pl.BlockSpec((1, H, C), lambda u, ex: (ex[u], 0, 0)),
            pl.BlockSpec((1, 1, C), lambda u, ex: (ex[u], 0, 0)),
        ],
        out_specs=[
            pl.BlockSpec((TB, C), lambda u, ex: (u, 0)),
            pl.BlockSpec((1, 1), lambda u, ex: (0, 0)),
        ],
    )
    return pl.pallas_call(
        _ffn_body,
        grid_spec=grid_spec,
        out_shape=(
            jax.ShapeDtypeStruct((NPAD, C), jnp.float32),
            jax.ShapeDtypeStruct((1, 1), jnp.float32),
        ),
    )(unit_expert, xs, scores_pad,
      W1, b1.reshape(E, 1, H), W2, b2.reshape(E, 1, C))


def _route(idx):
    onehot = (idx[:, None] == jnp.arange(E, dtype=jnp.int32)[None, :])
    onehot = onehot.astype(jnp.int32)
    ranks_all = jnp.cumsum(onehot, axis=0)
    counts = ranks_all[-1]
    rank = jnp.take_along_axis(ranks_all, idx[:, None], axis=1)[:, 0] - 1
    tiles = (counts + TB - 1) // TB
    tile_off = jnp.concatenate([jnp.zeros((1,), jnp.int32),
                                jnp.cumsum(tiles)]).astype(jnp.int32)
    pos = tile_off[idx] * TB + rank
    tok = jnp.arange(T, dtype=jnp.int32)
    src_ids = jnp.zeros((NPAD,), jnp.int32).at[pos].set(tok)
    dst_ids = jnp.full((NPAD,), T, jnp.int32).at[pos].set(tok)
    u = jnp.arange(NB, dtype=jnp.int32)
    unit_expert = jnp.sum(
        (u[:, None] >= tile_off[None, 1:]).astype(jnp.int32), axis=1
    )
    unit_expert = jnp.minimum(unit_expert, E - 1)
    return pos, src_ids, dst_ids, unit_expert


def kernel(x, Wred, wg, W1, b1, W2, b2):
    xf = x.reshape(T, C)
    scores, idx = _gating_tc(xf, Wred, wg)
    idx = idx[:, 0]
    pos, src_ids, dst_ids, unit_expert = _route(idx)
    scores_pad = jnp.zeros((NPAD, 1), jnp.float32).at[pos].set(scores)
    xs = _sc_gather(xf, src_ids)
    ys, total = _ffn_tc(xs, scores_pad, unit_expert, W1, b1, W2, b2)
    out = _sc_scatter(ys, dst_ids)[:T]
    return (out, total[0, 0])


# speedup vs baseline: 1.9089x; 1.9089x over previous
import jax
import jax.numpy as jnp
from jax import lax
from jax.experimental import pallas as pl
from jax.experimental.pallas import tpu as pltpu
from jax.experimental.pallas import tpu_sc as plsc

B, T, C, H, E = 1, 2048, 768, 3072, 8
TB = 128
NB = T // TB + E
NPAD = NB * TB
NW = 32
BPW = T // NW
NCHUNK = 4
CH = BPW // NCHUNK
CS = 512


def _gating_body(x_ref, wred_ref, wg_ref, score_ref, pos_ref, uex_ref):
    xf = x_ref[...]
    red = jnp.dot(xf, wred_ref[...].T,
                  preferred_element_type=jnp.float32)
    wg = wg_ref[...]
    norm = jnp.sqrt(jnp.sum(wg * wg, axis=1, keepdims=True))
    wg_s = wg * (1.5 / norm)
    n2 = jnp.sqrt(jnp.sum(wg_s * wg_s, axis=1, keepdims=True))
    wg_n = wg_s / jnp.maximum(n2, 1e-4)
    logits = jnp.dot(red, wg_n.T,
                     preferred_element_type=jnp.float32)
    lmax = jnp.max(logits, axis=1, keepdims=True)
    z = jnp.sum(jnp.exp(logits - lmax), axis=1, keepdims=True)
    score_ref[...] = 1.0 / z

    col = lax.broadcasted_iota(jnp.int32, logits.shape, 1)
    amax = jnp.min(jnp.where(logits >= lmax, col, jnp.int32(E)),
                   axis=1, keepdims=True)
    onehot = (col == amax).astype(jnp.float32)

    ri = lax.broadcasted_iota(jnp.int32, (CS, CS), 0)
    ci = lax.broadcasted_iota(jnp.int32, (CS, CS), 1)
    ltri = (ri >= ci).astype(jnp.float32)
    carry = jnp.zeros((1, E), jnp.float32)
    ranks_parts = []
    for i in range(T // CS):
        chunk = onehot[i * CS:(i + 1) * CS, :]
        ccum = jnp.dot(ltri, chunk, preferred_element_type=jnp.float32)
        ranks_parts.append(ccum + carry)
        carry = carry + ccum[CS - 1:CS, :]
    ranks_all = jnp.concatenate(ranks_parts, axis=0)
    counts = carry

    tiles = jnp.floor((counts + (TB - 1)) * (1.0 / TB))
    emask = (lax.broadcasted_iota(jnp.int32, (E, E), 0) <
             lax.broadcasted_iota(jnp.int32, (E, E), 1)).astype(jnp.float32)
    tile_off = jnp.dot(tiles, emask,
                       preferred_element_type=jnp.float32)

    rank = jnp.sum(onehot * ranks_all, axis=1, keepdims=True)
    base = jnp.dot(onehot, tile_off.T,
                   preferred_element_type=jnp.float32)
    pos_ref[...] = (base * TB + rank - 1.0).astype(jnp.int32)

    tile_end = tile_off + tiles
    uio = lax.broadcasted_iota(jnp.int32, (NB, 1), 0).astype(jnp.float32)
    uex = jnp.sum((uio >= tile_end).astype(jnp.float32), axis=1, keepdims=True)
    uex_ref[...] = jnp.minimum(uex, float(E - 1)).astype(jnp.int32)


def _gating_tc(xf, Wred, wg):
    return pl.pallas_call(
        _gating_body,
        out_shape=(
            jax.ShapeDtypeStruct((T, 1), jnp.float32),
            jax.ShapeDtypeStruct((T, 1), jnp.int32),
            jax.ShapeDtypeStruct((NB, 1), jnp.int32),
        ),
    )(xf, Wred, wg)


def _sc_scatter_body(x_hbm, pos_hbm, out_hbm, idx_v, rows_v, sem, semw):
    wid = lax.axis_index("s") * 2 + lax.axis_index("c")
    base = wid * BPW
    pltpu.sync_copy(pos_hbm.at[wid], idx_v)
    pltpu.async_copy(x_hbm.at[pl.ds(base, BPW)], rows_v, sem).wait()
    cps = [
        pltpu.async_copy(rows_v.at[pl.ds(k * CH, CH)],
                         out_hbm.at[idx_v.at[k]], semw)
        for k in range(NCHUNK)
    ]
    for c in cps:
        c.wait()


def _sc_scatter(xf, pos):
    mesh = plsc.VectorSubcoreMesh(core_axis_name="c", subcore_axis_name="s")
    return pl.kernel(
        _sc_scatter_body,
        out_type=jax.ShapeDtypeStruct((NPAD, C), jnp.float32),
        mesh=mesh,
        scratch_types=[
            pltpu.VMEM((NCHUNK, CH), jnp.int32),
            pltpu.VMEM((BPW, C), jnp.float32),
            pltpu.SemaphoreType.DMA,
            pltpu.SemaphoreType.DMA,
        ],
    )(xf, pos.reshape(NW, NCHUNK, CH))


def _sc_gather_body(ys_hbm, pos_hbm, out_hbm, idx_v, rows_v, sem):
    wid = lax.axis_index("s") * 2 + lax.axis_index("c")
    base = wid * BPW
    pltpu.sync_copy(pos_hbm.at[pl.ds(base, BPW)], idx_v)
    cps = [
        pltpu.async_copy(ys_hbm.at[idx_v.at[pl.ds(k * CH, CH)]],
                         rows_v.at[pl.ds(k * CH, CH)], sem)
        for k in range(NCHUNK)
    ]
    for c in cps:
        c.wait()
    pltpu.sync_copy(rows_v, out_hbm.at[pl.ds(base, BPW)])


def _sc_gather(ys, pos):
    mesh = plsc.VectorSubcoreMesh(core_axis_name="c", subcore_axis_name="s")
    return pl.kernel(
        _sc_gather_body,
        out_type=jax.ShapeDtypeStruct((T, C), jnp.float32),
        mesh=mesh,
        scratch_types=[
            pltpu.VMEM((BPW,), jnp.int32),
            pltpu.VMEM((BPW, C), jnp.float32),
            pltpu.SemaphoreType.DMA,
        ],
    )(ys, pos.reshape(T))


def _ffn_body(ex_ref, xs_ref, sc_ref, w1_ref, b1_ref, w2_ref, b2_ref,
              ys_ref, sum_ref):
    u = pl.program_id(0)
    xt = xs_ref[...]
    h = jnp.dot(xt, w1_ref[0], preferred_element_type=jnp.float32)
    h = h + b1_ref[0]
    h = 0.5 * h * (1.0 + lax.erf(h * 0.7071067811865476))
    y = jnp.dot(h, w2_ref[0], preferred_element_type=jnp.float32)
    sc = sc_ref[...]
    y = jnp.where(sc > 0.0, (y + b2_ref[0]) * sc, 0.0)
    ys_ref[...] = y

    @pl.when(u == 0)
    def _():
        sum_ref[...] = jnp.zeros((1, 1), jnp.float32)

    sum_ref[...] += jnp.sum(y).reshape(1, 1)


def _ffn_tc(xs, scores_pad, unit_expert, W1, b1, W2, b2):
    grid_spec = pltpu.PrefetchScalarGridSpec(
        num_scalar_prefetch=1,
        grid=(NB,),
        in_specs=[
            pl.BlockSpec((TB, C), lambda u, ex: (u, 0)),
            pl.BlockSpec((TB, 1), lambda u, ex: (u, 0)),
            pl.BlockSpec((1, C, H), lambda u, ex: (ex[u], 0, 0)),
            pl.BlockSpec((1, 1, H), lambda u, ex: (ex[u], 0, 0)),
            pl.BlockSpec((1, H, C), lambda u, ex: (ex[u], 0, 0)),
            pl.BlockSpec((1, 1, C), lambda u, ex: (ex[u], 0, 0)),
        ],
        out_specs=[
            pl.BlockSpec((TB, C), lambda u, ex: (u, 0)),
            pl.BlockSpec((1, 1), lambda u, ex: (0, 0)),
        ],
    )
    return pl.pallas_call(
        _ffn_body,
        grid_spec=grid_spec,
        out_shape=(
            jax.ShapeDtypeStruct((NPAD, C), jnp.float32),
            jax.ShapeDtypeStruct((1, 1), jnp.float32),
        ),
    )(unit_expert, xs, scores_pad,
      W1, b1.reshape(E, 1, H), W2, b2.reshape(E, 1, C))


def kernel(x, Wred, wg, W1, b1, W2, b2):
    xf = x.reshape(T, C)
    scores, pos, unit_expert = _gating_tc(xf, Wred, wg)
    scores_pad = jnp.zeros((NPAD,), jnp.float32).at[pos[:, 0]].set(
        scores[:, 0]).reshape(NPAD, 1)
    xs = _sc_scatter(xf, pos)
    ys, total = _ffn_tc(xs, scores_pad, unit_expert[:, 0], W1, b1, W2, b2)
    out = _sc_gather(ys, pos)
    return (out, total[0, 0])


# speedup vs baseline: 2.0051x; 1.0504x over previous
import jax
import jax.numpy as jnp
from jax import lax
from jax.experimental import pallas as pl
from jax.experimental.pallas import tpu as pltpu
from jax.experimental.pallas import tpu_sc as plsc

B, T, C, H, E = 1, 2048, 768, 3072, 8
TB = 128
NB = T // TB + E - 1
NPAD = NB * TB
NW = 32
BPW = T // NW
NCHUNK = 4
CH = BPW // NCHUNK
CS = 512


def _gating_body(x_ref, wred_ref, wg_ref, score_ref, pos_ref, uex_ref,
                 uval_ref):
    xf = x_ref[...]
    red = jnp.dot(xf, wred_ref[...].T,
                  preferred_element_type=jnp.float32)
    wg = wg_ref[...]
    norm = jnp.sqrt(jnp.sum(wg * wg, axis=1, keepdims=True))
    wg_s = wg * (1.5 / norm)
    n2 = jnp.sqrt(jnp.sum(wg_s * wg_s, axis=1, keepdims=True))
    wg_n = wg_s / jnp.maximum(n2, 1e-4)
    logits = jnp.dot(red, wg_n.T,
                     preferred_element_type=jnp.float32)
    lmax = jnp.max(logits, axis=1, keepdims=True)
    z = jnp.sum(jnp.exp(logits - lmax), axis=1, keepdims=True)
    score_ref[...] = 1.0 / z

    col = lax.broadcasted_iota(jnp.int32, logits.shape, 1)
    amax = jnp.min(jnp.where(logits >= lmax, col, jnp.int32(E)),
                   axis=1, keepdims=True)
    onehot = (col == amax).astype(jnp.float32)

    ri = lax.broadcasted_iota(jnp.int32, (CS, CS), 0)
    ci = lax.broadcasted_iota(jnp.int32, (CS, CS), 1)
    ltri = (ri >= ci).astype(jnp.float32)
    carry = jnp.zeros((1, E), jnp.float32)
    ranks_parts = []
    for i in range(T // CS):
        chunk = onehot[i * CS:(i + 1) * CS, :]
        ccum = jnp.dot(ltri, chunk, preferred_element_type=jnp.float32)
        ranks_parts.append(ccum + carry)
        carry = carry + ccum[CS - 1:CS, :]
    ranks_all = jnp.concatenate(ranks_parts, axis=0)
    counts = carry

    tiles = jnp.floor((counts + (TB - 1)) * (1.0 / TB))
    emask = (lax.broadcasted_iota(jnp.int32, (E, E), 0) <
             lax.broadcasted_iota(jnp.int32, (E, E), 1)).astype(jnp.float32)
    tile_off = jnp.dot(tiles, emask,
                       preferred_element_type=jnp.float32)

    rank = jnp.sum(onehot * ranks_all, axis=1, keepdims=True)
    base = jnp.dot(onehot, tile_off.T,
                   preferred_element_type=jnp.float32)
    pos_ref[...] = (base * TB + rank - 1.0).astype(jnp.int32)

    tile_end = tile_off + tiles
    total_tiles = jnp.sum(tiles)
    eio = lax.broadcasted_iota(jnp.int32, (1, E), 1).astype(jnp.float32)
    last_e = jnp.max(jnp.where(counts > 0.0, eio, 0.0))
    uio = lax.broadcasted_iota(jnp.int32, (NB, 1), 0).astype(jnp.float32)
    uex = jnp.sum((uio >= tile_end).astype(jnp.float32), axis=1, keepdims=True)
    uex_ref[...] = jnp.minimum(uex, last_e).astype(jnp.int32)
    uval_ref[...] = (uio < total_tiles).astype(jnp.int32)


def _gating_tc(xf, Wred, wg):
    return pl.pallas_call(
        _gating_body,
        out_shape=(
            jax.ShapeDtypeStruct((T, 1), jnp.float32),
            jax.ShapeDtypeStruct((T, 1), jnp.int32),
            jax.ShapeDtypeStruct((NB, 1), jnp.int32),
            jax.ShapeDtypeStruct((NB, 1), jnp.int32),
        ),
    )(xf, Wred, wg)


def _sc_scatter_body(x_hbm, pos_hbm, out_hbm, idx_v, rows_v, sem, semw):
    wid = lax.axis_index("s") * 2 + lax.axis_index("c")
    base = wid * BPW
    pltpu.sync_copy(pos_hbm.at[wid], idx_v)
    pltpu.async_copy(x_hbm.at[pl.ds(base, BPW)], rows_v, sem).wait()
    cps = [
        pltpu.async_copy(rows_v.at[pl.ds(k * CH, CH)],
                         out_hbm.at[idx_v.at[k]], semw)
        for k in range(NCHUNK)
    ]
    for c in cps:
        c.wait()


def _sc_scatter(xf, pos):
    mesh = plsc.VectorSubcoreMesh(core_axis_name="c", subcore_axis_name="s")
    return pl.kernel(
        _sc_scatter_body,
        out_type=jax.ShapeDtypeStruct((NPAD, C), jnp.float32),
        mesh=mesh,
        scratch_types=[
            pltpu.VMEM((NCHUNK, CH), jnp.int32),
            pltpu.VMEM((BPW, C), jnp.float32),
            pltpu.SemaphoreType.DMA,
            pltpu.SemaphoreType.DMA,
        ],
    )(xf, pos.reshape(NW, NCHUNK, CH))


def _sc_gather_body(ys_hbm, pos_hbm, out_hbm, idx_v, rows_v, sem):
    wid = lax.axis_index("s") * 2 + lax.axis_index("c")
    base = wid * BPW
    pltpu.sync_copy(pos_hbm.at[pl.ds(base, BPW)], idx_v)
    cps = [
        pltpu.async_copy(ys_hbm.at[idx_v.at[pl.ds(k * CH, CH)]],
                         rows_v.at[pl.ds(k * CH, CH)], sem)
        for k in range(NCHUNK)
    ]
    for c in cps:
        c.wait()
    pltpu.sync_copy(rows_v, out_hbm.at[pl.ds(base, BPW)])


def _sc_gather(ys, pos):
    mesh = plsc.VectorSubcoreMesh(core_axis_name="c", subcore_axis_name="s")
    return pl.kernel(
        _sc_gather_body,
        out_type=jax.ShapeDtypeStruct((T, C), jnp.float32),
        mesh=mesh,
        scratch_types=[
            pltpu.VMEM((BPW,), jnp.int32),
            pltpu.VMEM((BPW, C), jnp.float32),
            pltpu.SemaphoreType.DMA,
        ],
    )(ys, pos.reshape(T))


def _ffn_body(ex_ref, uv_ref, xs_ref, sc_ref, w1_ref, b1_ref, w2_ref, b2_ref,
              ys_ref, sum_ref):
    u = pl.program_id(0)

    @pl.when(u == 0)
    def _():
        sum_ref[...] = jnp.zeros((1, 1), jnp.float32)

    @pl.when(uv_ref[u] == 1)
    def _():
        xt = xs_ref[...]
        h = jnp.dot(xt, w1_ref[0], preferred_element_type=jnp.float32)
        h = h + b1_ref[0]
        h = 0.5 * h * (1.0 + lax.erf(h * 0.7071067811865476))
        y = jnp.dot(h, w2_ref[0], preferred_element_type=jnp.float32)
        sc = sc_ref[...]
        y = jnp.where(sc > 0.0, (y + b2_ref[0]) * sc, 0.0)
        ys_ref[...] = y
        sum_ref[...] += jnp.sum(y).reshape(1, 1)


def _ffn_tc(xs, scores_pad, unit_expert, unit_valid, W1, b1, W2, b2):
    grid_spec = pltpu.PrefetchScalarGridSpec(
        num_scalar_prefetch=2,
        grid=(NB,),
        in_specs=[
            pl.BlockSpec((TB, C), lambda u, ex, uv: (u, 0)),
            pl.BlockSpec((TB, 1), lambda u, ex, uv: (u, 0)),
            pl.BlockSpec((1, C, H), lambda u, ex, uv: (ex[u], 0, 0)),
            pl.BlockSpec((1, 1, H), lambda u, ex, uv: (ex[u], 0, 0)),
            pl.BlockSpec((1, H, C), lambda u, ex, uv: (ex[u], 0, 0)),
            pl.BlockSpec((1, 1, C), lambda u, ex, uv: (ex[u], 0, 0)),
        ],
        out_specs=[
            pl.BlockSpec((TB, C), lambda u, ex, uv: (u, 0)),
            pl.BlockSpec((1, 1), lambda u, ex, uv: (0, 0)),
        ],
    )
    return pl.pallas_call(
        _ffn_body,
        grid_spec=grid_spec,
        out_shape=(
            jax.ShapeDtypeStruct((NPAD, C), jnp.float32),
            jax.ShapeDtypeStruct((1, 1), jnp.float32),
        ),
    )(unit_expert, unit_valid, xs, scores_pad,
      W1, b1.reshape(E, 1, H), W2, b2.reshape(E, 1, C))


def kernel(x, Wred, wg, W1, b1, W2, b2):
    xf = x.reshape(T, C)
    scores, pos, unit_expert, unit_valid = _gating_tc(xf, Wred, wg)
    scores_pad = jnp.zeros((NPAD,), jnp.float32).at[pos[:, 0]].set(
        scores[:, 0]).reshape(NPAD, 1)
    xs = _sc_scatter(xf, pos)
    ys, total = _ffn_tc(xs, scores_pad, unit_expert[:, 0], unit_valid[:, 0],
                        W1, b1, W2, b2)
    out = _sc_gather(ys, pos)
    return (out, total[0, 0])
